# Initial kernel scaffold; baseline (speedup 1.0000x reference)
import jax
import jax.numpy as jnp
from jax.experimental import pallas as pl


def kernel(z, edge_index):
    raise NotImplementedError("write your pallas kernel here")


# speedup vs baseline: 3.4574x; 3.4574x over previous
import dataclasses
import functools

import jax
import jax.numpy as jnp
from jax import lax
from jax.experimental import pallas as pl
from jax.experimental.pallas import tpu as pltpu
from jax.experimental.pallas import tpu_sc as plsc

_D = 128
_W = 128
_NC = 2
_NS = 16
_NW = _NC * _NS
_L = 16


def _sc_dot_gather(z, src, dst, n_edges):
    n_chunks = n_edges // _W
    mesh = plsc.VectorSubcoreMesh(core_axis_name="c", subcore_axis_name="s")
    cp = pltpu.CompilerParams()
    if "needs_layout_passes" in pltpu.CompilerParams.__dataclass_fields__:
        cp = dataclasses.replace(cp, needs_layout_passes=False)

    @functools.partial(
        pl.kernel,
        out_type=jax.ShapeDtypeStruct((n_edges,), jnp.float32),
        mesh=mesh,
        compiler_params=cp,
        scratch_types=[
            pltpu.VMEM((_W,), jnp.int32),
            pltpu.VMEM((_W,), jnp.int32),
            pltpu.VMEM((_W, _D), jnp.float32),
            pltpu.VMEM((_W, _D), jnp.float32),
            pltpu.VMEM((_W * _L,), jnp.float32),
            pltpu.VMEM((_W,), jnp.float32),
        ],
    )
    def k(z_hbm, src_hbm, dst_hbm, out_hbm, sidx, didx, srows, drows, part, outv):
        wid = lax.axis_index("s") * _NC + lax.axis_index("c")
        coloffs = lax.iota(jnp.int32, _L) * _L

        @pl.loop(wid, n_chunks, step=_NW)
        def _(c):
            base = c * _W
            pltpu.sync_copy(src_hbm.at[pl.ds(base, _W)], sidx)
            pltpu.sync_copy(dst_hbm.at[pl.ds(base, _W)], didx)
            pltpu.sync_copy(z_hbm.at[sidx], srows)
            pltpu.sync_copy(z_hbm.at[didx], drows)

            @pl.loop(0, _W)
            def _(r):
                acc = srows[r, pl.ds(0, _L)] * drows[r, pl.ds(0, _L)]
                for kk in range(1, _D // _L):
                    acc = acc + (srows[r, pl.ds(kk * _L, _L)]
                                 * drows[r, pl.ds(kk * _L, _L)])
                part[pl.ds(r * _L, _L)] = acc

            @pl.loop(0, _W // _L)
            def _(g):
                red = plsc.load_gather(part, [coloffs + g * (_L * _L)])
                for kk in range(1, _L):
                    red = red + plsc.load_gather(
                        part, [coloffs + (g * (_L * _L) + kk)])
                outv[pl.ds(g * _L, _L)] = red

            pltpu.sync_copy(outv, out_hbm.at[pl.ds(base, _W)])

    return k(z, src, dst)


def kernel(z, edge_index):
    n_edges = edge_index.shape[1]
    src = edge_index[0].astype(jnp.int32)
    dst = edge_index[1].astype(jnp.int32)
    return _sc_dot_gather(z, src, dst, n_edges)


# speedup vs baseline: 7.3559x; 2.1276x over previous
import dataclasses
import functools

import jax
import jax.numpy as jnp
from jax import lax
from jax.experimental import pallas as pl
from jax.experimental.pallas import tpu as pltpu
from jax.experimental.pallas import tpu_sc as plsc

_D = 128
_W = 80
_NC = 2
_NS = 16
_NW = _NC * _NS
_L = 16


def _sc_dot_gather(z, src, dst, n_edges):
    epw = n_edges // _NW
    n_ch = epw // _W
    mesh = plsc.VectorSubcoreMesh(core_axis_name="c", subcore_axis_name="s")
    cp = pltpu.CompilerParams()
    if "needs_layout_passes" in pltpu.CompilerParams.__dataclass_fields__:
        cp = dataclasses.replace(cp, needs_layout_passes=False)

    @functools.partial(
        pl.kernel,
        out_type=jax.ShapeDtypeStruct((n_edges,), jnp.float32),
        mesh=mesh,
        compiler_params=cp,
        scratch_types=[
            pltpu.VMEM((epw,), jnp.int32),
            pltpu.VMEM((epw,), jnp.int32),
            pltpu.VMEM((_W, _D), jnp.float32),
            pltpu.VMEM((_W, _D), jnp.float32),
            pltpu.VMEM((_W, _D), jnp.float32),
            pltpu.VMEM((_W, _D), jnp.float32),
            pltpu.VMEM((_W * _L,), jnp.float32),
            pltpu.VMEM((epw,), jnp.float32),
            pltpu.SemaphoreType.DMA,
            pltpu.SemaphoreType.DMA,
            pltpu.SemaphoreType.DMA,
        ],
    )
    def k(z_hbm, src_hbm, dst_hbm, out_hbm, sidx, didx, srows0, srows1,
          drows0, drows1, part, outa, gsem0, gsem1, isem):
        wid = lax.axis_index("s") * _NC + lax.axis_index("c")
        base0 = wid * epw
        coloffs = lax.iota(jnp.int32, _L) * _L

        c1 = pltpu.async_copy(src_hbm.at[pl.ds(base0, epw)], sidx, isem)
        c2 = pltpu.async_copy(dst_hbm.at[pl.ds(base0, epw)], didx, isem)
        c1.wait()
        c2.wait()

        def issue(c, sbuf, dbuf, sem):
            off = c * _W
            pltpu.async_copy(z_hbm.at[sidx.at[pl.ds(off, _W)]], sbuf, sem)
            pltpu.async_copy(z_hbm.at[didx.at[pl.ds(off, _W)]], dbuf, sem)

        def drain(c, sbuf, dbuf, sem):
            off = c * _W
            pltpu.make_async_copy(
                z_hbm.at[sidx.at[pl.ds(off, _W)]], sbuf, sem).wait()
            pltpu.make_async_copy(
                z_hbm.at[didx.at[pl.ds(off, _W)]], dbuf, sem).wait()

        def compute(c, sbuf, dbuf):
            @pl.loop(0, _W)
            def _(r):
                acc = sbuf[r, pl.ds(0, _L)] * dbuf[r, pl.ds(0, _L)]
                for kk in range(1, _D // _L):
                    acc = acc + (sbuf[r, pl.ds(kk * _L, _L)]
                                 * dbuf[r, pl.ds(kk * _L, _L)])
                part[pl.ds(r * _L, _L)] = acc

            @pl.loop(0, _W // _L)
            def _(g):
                red = plsc.load_gather(part, [coloffs + g * (_L * _L)])
                for kk in range(1, _L):
                    red = red + plsc.load_gather(
                        part, [coloffs + (g * (_L * _L) + kk)])
                outa[pl.ds(c * _W + g * _L, _L)] = red

        issue(0, srows0, drows0, gsem0)

        @pl.loop(0, n_ch - 1, step=2)
        def _(i):
            drain(i, srows0, drows0, gsem0)
            issue(i + 1, srows1, drows1, gsem1)
            compute(i, srows0, drows0)
            drain(i + 1, srows1, drows1, gsem1)
            issue(i + 2, srows0, drows0, gsem0)
            compute(i + 1, srows1, drows1)

        drain(n_ch - 1, srows0, drows0, gsem0)
        compute(n_ch - 1, srows0, drows0)

        pltpu.sync_copy(outa, out_hbm.at[pl.ds(base0, epw)])

    return k(z, src, dst)


def kernel(z, edge_index):
    n_edges = edge_index.shape[1]
    src = edge_index[0].astype(jnp.int32)
    dst = edge_index[1].astype(jnp.int32)
    return _sc_dot_gather(z, src, dst, n_edges)


# speedup vs baseline: 7.9172x; 1.0763x over previous
import dataclasses
import functools

import jax
import jax.numpy as jnp
from jax import lax
from jax.experimental import pallas as pl
from jax.experimental.pallas import tpu as pltpu
from jax.experimental.pallas import tpu_sc as plsc

_D = 128
_W = 80
_NC = 2
_NS = 16
_NW = _NC * _NS
_L = 16
_LB = 32


def _sc_dot_gather(z_u32, src, dst, n_edges):
    epw = n_edges // _NW
    n_ch = epw // _W
    mesh = plsc.VectorSubcoreMesh(core_axis_name="c", subcore_axis_name="s")
    cp = pltpu.CompilerParams()
    if "needs_layout_passes" in pltpu.CompilerParams.__dataclass_fields__:
        cp = dataclasses.replace(cp, needs_layout_passes=False)
    if "use_tc_tiling_on_sc" in pltpu.CompilerParams.__dataclass_fields__:
        cp = dataclasses.replace(cp, use_tc_tiling_on_sc=False)

    @functools.partial(
        pl.kernel,
        out_type=jax.ShapeDtypeStruct((n_edges,), jnp.float32),
        mesh=mesh,
        compiler_params=cp,
        scratch_types=[
            pltpu.VMEM((epw,), jnp.int32),
            pltpu.VMEM((epw,), jnp.int32),
            pltpu.VMEM((_W, _D // 2), jnp.uint32),
            pltpu.VMEM((_W, _D // 2), jnp.uint32),
            pltpu.VMEM((_W, _D // 2), jnp.uint32),
            pltpu.VMEM((_W, _D // 2), jnp.uint32),
            pltpu.VMEM((_W * _L,), jnp.float32),
            pltpu.VMEM((epw,), jnp.float32),
            pltpu.SemaphoreType.DMA,
            pltpu.SemaphoreType.DMA,
            pltpu.SemaphoreType.DMA,
        ],
    )
    def k(z_hbm, src_hbm, dst_hbm, out_hbm, sidx, didx, srows0, srows1,
          drows0, drows1, part, outa, gsem0, gsem1, isem):
        wid = lax.axis_index("s") * _NC + lax.axis_index("c")
        base0 = wid * epw
        coloffs = lax.iota(jnp.int32, _L) * _L

        c1 = pltpu.async_copy(src_hbm.at[pl.ds(base0, epw)], sidx, isem)
        c2 = pltpu.async_copy(dst_hbm.at[pl.ds(base0, epw)], didx, isem)
        c1.wait()
        c2.wait()

        def issue(c, sbuf, dbuf, sem):
            off = c * _W
            pltpu.async_copy(z_hbm.at[sidx.at[pl.ds(off, _W)]], sbuf, sem)
            pltpu.async_copy(z_hbm.at[didx.at[pl.ds(off, _W)]], dbuf, sem)

        def drain(c, sbuf, dbuf, sem):
            off = c * _W
            pltpu.make_async_copy(
                z_hbm.at[sidx.at[pl.ds(off, _W)]], sbuf, sem).wait()
            pltpu.make_async_copy(
                z_hbm.at[didx.at[pl.ds(off, _W)]], dbuf, sem).wait()

        def compute(c, sbuf, dbuf):
            @pl.loop(0, _W)
            def _(r):
                acc = None
                for kk in range(_D // _LB):
                    sv = plsc.bitcast(sbuf[r, pl.ds(kk * _L, _L)],
                                      jnp.bfloat16)
                    dv = plsc.bitcast(dbuf[r, pl.ds(kk * _L, _L)],
                                      jnp.bfloat16)
                    p = sv * dv
                    lo, hi = plsc.unpack(p, format=plsc.PackFormat.INTERLEAVED)
                    s = lo + hi
                    acc = s if acc is None else acc + s
                part[pl.ds(r * _L, _L)] = acc

            @pl.loop(0, _W // _L)
            def _(g):
                red = plsc.load_gather(part, [coloffs + g * (_L * _L)])
                for kk in range(1, _L):
                    red = red + plsc.load_gather(
                        part, [coloffs + (g * (_L * _L) + kk)])
                outa[pl.ds(c * _W + g * _L, _L)] = red

        issue(0, srows0, drows0, gsem0)

        @pl.loop(0, n_ch - 1, step=2)
        def _(i):
            drain(i, srows0, drows0, gsem0)
            issue(i + 1, srows1, drows1, gsem1)
            compute(i, srows0, drows0)
            drain(i + 1, srows1, drows1, gsem1)
            issue(i + 2, srows0, drows0, gsem0)
            compute(i + 1, srows1, drows1)

        drain(n_ch - 1, srows0, drows0, gsem0)
        compute(n_ch - 1, srows0, drows0)

        pltpu.sync_copy(outa, out_hbm.at[pl.ds(base0, epw)])

    return k(z_u32, src, dst)


def kernel(z, edge_index):
    n_edges = edge_index.shape[1]
    src = edge_index[0].astype(jnp.int32)
    dst = edge_index[1].astype(jnp.int32)
    z_u32 = jax.lax.bitcast_convert_type(
        z.astype(jnp.bfloat16).reshape(z.shape[0], z.shape[1] // 2, 2),
        jnp.uint32)
    return _sc_dot_gather(z_u32, src, dst, n_edges)


# speedup vs baseline: 8.4951x; 1.0730x over previous
import dataclasses
import functools

import jax
import jax.numpy as jnp
from jax import lax
from jax.experimental import pallas as pl
from jax.experimental.pallas import tpu as pltpu
from jax.experimental.pallas import tpu_sc as plsc

_D = 128
_W = 80
_NC = 2
_NS = 16
_NW = _NC * _NS
_L = 16
_LB = 32


def _sc_dot_gather(z_u32, src, dst, n_edges):
    epw = n_edges // _NW
    n_ch = epw // _W
    mesh = plsc.VectorSubcoreMesh(core_axis_name="c", subcore_axis_name="s")
    cp = pltpu.CompilerParams()
    if "needs_layout_passes" in pltpu.CompilerParams.__dataclass_fields__:
        cp = dataclasses.replace(cp, needs_layout_passes=False)
    if "use_tc_tiling_on_sc" in pltpu.CompilerParams.__dataclass_fields__:
        cp = dataclasses.replace(cp, use_tc_tiling_on_sc=False)

    @functools.partial(
        pl.kernel,
        out_type=jax.ShapeDtypeStruct((n_edges,), jnp.float32),
        mesh=mesh,
        compiler_params=cp,
        scratch_types=[
            pltpu.VMEM((epw,), jnp.int32),
            pltpu.VMEM((epw,), jnp.int32),
            pltpu.VMEM((_W, _D // 2), jnp.uint32),
            pltpu.VMEM((_W, _D // 2), jnp.uint32),
            pltpu.VMEM((_W, _D // 2), jnp.uint32),
            pltpu.VMEM((_W, _D // 2), jnp.uint32),
            pltpu.VMEM((_W * _L,), jnp.float32),
            pltpu.VMEM((epw,), jnp.float32),
            pltpu.SemaphoreType.DMA,
            pltpu.SemaphoreType.DMA,
            pltpu.SemaphoreType.DMA,
        ],
    )
    def k(z_hbm, src_hbm, dst_hbm, out_hbm, sidx, didx, srows0, srows1,
          drows0, drows1, part, outa, gsem0, gsem1, isem):
        wid = lax.axis_index("s") * _NC + lax.axis_index("c")
        base0 = wid * epw
        coloffs = lax.iota(jnp.int32, _L) * _L

        c1 = pltpu.async_copy(src_hbm.at[pl.ds(base0, epw)], sidx, isem)
        c2 = pltpu.async_copy(dst_hbm.at[pl.ds(base0, epw)], didx, isem)
        c1.wait()
        c2.wait()

        def issue(c, sbuf, dbuf, sem):
            off = c * _W
            pltpu.async_copy(z_hbm.at[sidx.at[pl.ds(off, _W)]], sbuf, sem)
            pltpu.async_copy(z_hbm.at[didx.at[pl.ds(off, _W)]], dbuf, sem)

        def drain(c, sbuf, dbuf, sem):
            off = c * _W
            pltpu.make_async_copy(
                z_hbm.at[sidx.at[pl.ds(off, _W)]], sbuf, sem).wait()
            pltpu.make_async_copy(
                z_hbm.at[didx.at[pl.ds(off, _W)]], dbuf, sem).wait()

        def compute(c, sbuf, dbuf):
            @plsc.parallel_loop(0, _W, unroll=4)
            def _(r):
                acc = None
                for kk in range(_D // _LB):
                    sv = plsc.bitcast(sbuf[r, pl.ds(kk * _L, _L)],
                                      jnp.bfloat16)
                    dv = plsc.bitcast(dbuf[r, pl.ds(kk * _L, _L)],
                                      jnp.bfloat16)
                    p = sv * dv
                    lo, hi = plsc.unpack(p, format=plsc.PackFormat.INTERLEAVED)
                    s = lo + hi
                    acc = s if acc is None else acc + s
                part[pl.ds(r * _L, _L)] = acc

            @plsc.parallel_loop(0, _W // _L, unroll=5)
            def _(g):
                red = plsc.load_gather(part, [coloffs + g * (_L * _L)])
                for kk in range(1, _L):
                    red = red + plsc.load_gather(
                        part, [coloffs + (g * (_L * _L) + kk)])
                outa[pl.ds(c * _W + g * _L, _L)] = red

        issue(0, srows0, drows0, gsem0)

        @pl.loop(0, n_ch - 1, step=2)
        def _(i):
            drain(i, srows0, drows0, gsem0)
            issue(i + 1, srows1, drows1, gsem1)
            compute(i, srows0, drows0)
            drain(i + 1, srows1, drows1, gsem1)
            issue(i + 2, srows0, drows0, gsem0)
            compute(i + 1, srows1, drows1)

        drain(n_ch - 1, srows0, drows0, gsem0)
        compute(n_ch - 1, srows0, drows0)

        pltpu.sync_copy(outa, out_hbm.at[pl.ds(base0, epw)])

    return k(z_u32, src, dst)


def kernel(z, edge_index):
    n_edges = edge_index.shape[1]
    src = edge_index[0].astype(jnp.int32)
    dst = edge_index[1].astype(jnp.int32)
    z_u32 = jax.lax.bitcast_convert_type(
        z.astype(jnp.bfloat16).reshape(z.shape[0], z.shape[1] // 2, 2),
        jnp.uint32)
    return _sc_dot_gather(z_u32, src, dst, n_edges)


# speedup vs baseline: 8.5001x; 1.0006x over previous
import dataclasses
import functools

import jax
import jax.numpy as jnp
from jax import lax
from jax.experimental import pallas as pl
from jax.experimental.pallas import tpu as pltpu
from jax.experimental.pallas import tpu_sc as plsc

_D = 128
_W = 80
_NC = 2
_NS = 16
_NW = _NC * _NS
_L = 16
_LB = 32


def _sc_dot_gather(z_u32, src, dst, n_edges):
    epw = n_edges // _NW
    n_ch = epw // _W
    mesh = plsc.VectorSubcoreMesh(core_axis_name="c", subcore_axis_name="s")
    cp = pltpu.CompilerParams()
    if "needs_layout_passes" in pltpu.CompilerParams.__dataclass_fields__:
        cp = dataclasses.replace(cp, needs_layout_passes=False)
    if "use_tc_tiling_on_sc" in pltpu.CompilerParams.__dataclass_fields__:
        cp = dataclasses.replace(cp, use_tc_tiling_on_sc=False)

    @functools.partial(
        pl.kernel,
        out_type=jax.ShapeDtypeStruct((n_edges,), jnp.float32),
        mesh=mesh,
        compiler_params=cp,
        scratch_types=[
            pltpu.VMEM((epw,), jnp.int32),
            pltpu.VMEM((epw,), jnp.int32),
            pltpu.VMEM((_W, _D // 2), jnp.uint32),
            pltpu.VMEM((_W, _D // 2), jnp.uint32),
            pltpu.VMEM((_W, _D // 2), jnp.uint32),
            pltpu.VMEM((_W, _D // 2), jnp.uint32),
            pltpu.VMEM((_W * _L,), jnp.float32),
            pltpu.VMEM((epw,), jnp.float32),
            pltpu.SemaphoreType.DMA,
            pltpu.SemaphoreType.DMA,
            pltpu.SemaphoreType.DMA,
        ],
    )
    def k(z_hbm, src_hbm, dst_hbm, out_hbm, sidx, didx, srows0, srows1,
          drows0, drows1, part, outa, gsem0, gsem1, isem):
        wid = lax.axis_index("s") * _NC + lax.axis_index("c")
        base0 = wid * epw
        coloffs = lax.iota(jnp.int32, _L) * _L

        c1 = pltpu.async_copy(src_hbm.at[pl.ds(base0, epw)], sidx, isem)
        c2 = pltpu.async_copy(dst_hbm.at[pl.ds(base0, epw)], didx, isem)
        c1.wait()
        c2.wait()

        def issue(c, sbuf, dbuf, sem):
            off = c * _W
            pltpu.async_copy(z_hbm.at[sidx.at[pl.ds(off, _W)]], sbuf, sem)
            pltpu.async_copy(z_hbm.at[didx.at[pl.ds(off, _W)]], dbuf, sem)

        def drain(c, sbuf, dbuf, sem):
            off = c * _W
            pltpu.make_async_copy(
                z_hbm.at[sidx.at[pl.ds(off, _W)]], sbuf, sem).wait()
            pltpu.make_async_copy(
                z_hbm.at[didx.at[pl.ds(off, _W)]], dbuf, sem).wait()

        def compute(c, sbuf, dbuf):
            @plsc.parallel_loop(0, _W, unroll=8)
            def _(r):
                acc = None
                for kk in range(_D // _LB):
                    sv = plsc.bitcast(sbuf[r, pl.ds(kk * _L, _L)],
                                      jnp.bfloat16)
                    dv = plsc.bitcast(dbuf[r, pl.ds(kk * _L, _L)],
                                      jnp.bfloat16)
                    p = sv * dv
                    lo, hi = plsc.unpack(p, format=plsc.PackFormat.INTERLEAVED)
                    s = lo + hi
                    acc = s if acc is None else acc + s
                part[pl.ds(r * _L, _L)] = acc

            @plsc.parallel_loop(0, _W // _L, unroll=5)
            def _(g):
                red = plsc.load_gather(part, [coloffs + g * (_L * _L)])
                for kk in range(1, _L):
                    red = red + plsc.load_gather(
                        part, [coloffs + (g * (_L * _L) + kk)])
                outa[pl.ds(c * _W + g * _L, _L)] = red

        issue(0, srows0, drows0, gsem0)

        @pl.loop(0, n_ch - 1, step=2)
        def _(i):
            drain(i, srows0, drows0, gsem0)
            issue(i + 1, srows1, drows1, gsem1)
            compute(i, srows0, drows0)
            drain(i + 1, srows1, drows1, gsem1)
            issue(i + 2, srows0, drows0, gsem0)
            compute(i + 1, srows1, drows1)

        drain(n_ch - 1, srows0, drows0, gsem0)
        compute(n_ch - 1, srows0, drows0)

        pltpu.sync_copy(outa, out_hbm.at[pl.ds(base0, epw)])

    return k(z_u32, src, dst)


def kernel(z, edge_index):
    n_edges = edge_index.shape[1]
    src = edge_index[0].astype(jnp.int32)
    dst = edge_index[1].astype(jnp.int32)
    z_u32 = jax.lax.bitcast_convert_type(
        z.astype(jnp.bfloat16).reshape(z.shape[0], z.shape[1] // 2, 2),
        jnp.uint32)
    return _sc_dot_gather(z_u32, src, dst, n_edges)


# speedup vs baseline: 11.7103x; 1.3777x over previous
import dataclasses
import functools

import jax
import jax.numpy as jnp
from jax import lax
from jax.experimental import pallas as pl
from jax.experimental.pallas import tpu as pltpu
from jax.experimental.pallas import tpu_sc as plsc

_D = 128
_W = 80
_NC = 2
_NS = 16
_NW = _NC * _NS
_L = 16
_LB = 32


def _sc_dot_gather(z_u32, src, dst, n_edges):
    epw = n_edges // _NW
    n_ch = epw // _W
    mesh = plsc.VectorSubcoreMesh(core_axis_name="c", subcore_axis_name="s")
    cp = pltpu.CompilerParams()
    if "needs_layout_passes" in pltpu.CompilerParams.__dataclass_fields__:
        cp = dataclasses.replace(cp, needs_layout_passes=False)
    if "use_tc_tiling_on_sc" in pltpu.CompilerParams.__dataclass_fields__:
        cp = dataclasses.replace(cp, use_tc_tiling_on_sc=False)

    @functools.partial(
        pl.kernel,
        out_type=jax.ShapeDtypeStruct((n_edges,), jnp.float32),
        mesh=mesh,
        compiler_params=cp,
        scratch_types=[
            pltpu.VMEM((epw,), jnp.int32),
            pltpu.VMEM((epw,), jnp.int32),
            pltpu.VMEM((_W, _D // 2), jnp.uint32),
            pltpu.VMEM((_W, _D // 2), jnp.uint32),
            pltpu.VMEM((_W, _D // 2), jnp.uint32),
            pltpu.VMEM((_W, _D // 2), jnp.uint32),
            pltpu.VMEM((_W, _D // 2), jnp.uint32),
            pltpu.VMEM((_W, _D // 2), jnp.uint32),
            pltpu.VMEM((_W, _D // 2), jnp.uint32),
            pltpu.VMEM((_W, _D // 2), jnp.uint32),
            pltpu.VMEM((_W * _L,), jnp.float32),
            pltpu.VMEM((epw,), jnp.float32),
            pltpu.SemaphoreType.DMA,
            pltpu.SemaphoreType.DMA,
            pltpu.SemaphoreType.DMA,
            pltpu.SemaphoreType.DMA,
            pltpu.SemaphoreType.DMA,
        ],
    )
    def k(z_hbm, src_hbm, dst_hbm, out_hbm, sidx, didx, srows0, srows1,
          srows2, srows3, drows0, drows1, drows2, drows3, part, outa,
          gsem0, gsem1, gsem2, gsem3, isem):
        wid = lax.axis_index("s") * _NC + lax.axis_index("c")
        base0 = wid * epw
        coloffs = lax.iota(jnp.int32, _L) * _L

        c1 = pltpu.async_copy(src_hbm.at[pl.ds(base0, epw)], sidx, isem)
        c2 = pltpu.async_copy(dst_hbm.at[pl.ds(base0, epw)], didx, isem)
        c1.wait()
        c2.wait()

        def issue(c, sbuf, dbuf, sem):
            off = c * _W
            pltpu.async_copy(z_hbm.at[sidx.at[pl.ds(off, _W)]], sbuf, sem)
            pltpu.async_copy(z_hbm.at[didx.at[pl.ds(off, _W)]], dbuf, sem)

        def drain(c, sbuf, dbuf, sem):
            off = c * _W
            pltpu.make_async_copy(
                z_hbm.at[sidx.at[pl.ds(off, _W)]], sbuf, sem).wait()
            pltpu.make_async_copy(
                z_hbm.at[didx.at[pl.ds(off, _W)]], dbuf, sem).wait()

        def compute(c, sbuf, dbuf):
            @plsc.parallel_loop(0, _W, unroll=8)
            def _(r):
                acc = None
                for kk in range(_D // _LB):
                    sv = plsc.bitcast(sbuf[r, pl.ds(kk * _L, _L)],
                                      jnp.bfloat16)
                    dv = plsc.bitcast(dbuf[r, pl.ds(kk * _L, _L)],
                                      jnp.bfloat16)
                    p = sv * dv
                    lo, hi = plsc.unpack(p, format=plsc.PackFormat.INTERLEAVED)
                    s = lo + hi
                    acc = s if acc is None else acc + s
                part[pl.ds(r * _L, _L)] = acc

            @plsc.parallel_loop(0, _W // _L, unroll=5)
            def _(g):
                red = plsc.load_gather(part, [coloffs + g * (_L * _L)])
                for kk in range(1, _L):
                    red = red + plsc.load_gather(
                        part, [coloffs + (g * (_L * _L) + kk)])
                outa[pl.ds(c * _W + g * _L, _L)] = red

        slots = ((srows0, drows0, gsem0), (srows1, drows1, gsem1),
                 (srows2, drows2, gsem2), (srows3, drows3, gsem3))
        n_slots = len(slots)

        for b in range(n_slots - 1):
            issue(b, *slots[b])

        @pl.loop(0, n_ch - 1, step=n_slots)
        def _(i):
            for b in range(n_slots):
                sbuf, dbuf, sem = slots[b]
                nxt = i + b + (n_slots - 1)

                @pl.when(nxt <= n_ch - 1)
                def _():
                    sb, db, sm = slots[(b + n_slots - 1) % n_slots]
                    issue(nxt, sb, db, sm)

                drain(i + b, sbuf, dbuf, sem)
                compute(i + b, sbuf, dbuf)

        drain(n_ch - 1, srows0, drows0, gsem0)
        compute(n_ch - 1, srows0, drows0)

        pltpu.sync_copy(outa, out_hbm.at[pl.ds(base0, epw)])

    return k(z_u32, src, dst)


def kernel(z, edge_index):
    n_edges = edge_index.shape[1]
    src = edge_index[0].astype(jnp.int32)
    dst = edge_index[1].astype(jnp.int32)
    z_u32 = jax.lax.bitcast_convert_type(
        z.astype(jnp.bfloat16).reshape(z.shape[0], z.shape[1] // 2, 2),
        jnp.uint32)
    return _sc_dot_gather(z_u32, src, dst, n_edges)
